# baseline (speedup 1.0000x reference)
import functools

import jax
import jax.numpy as jnp
from jax import lax
from jax.experimental import pallas as pl
from jax.experimental.pallas import tpu as pltpu
from jax.experimental.pallas import tpu_sc as plsc

_K = 8192
_D = 256
_N = 9216
_TB = 768
_KB = 1024
_NT = _N // _TB
_NK = _K // _KB
_CC = 0.25

_NW = 32
_BPW = _N // _NW
_CH = 96
_NCH = _BPW // _CH


def _dist_body(x_ref, w_ref, idx_ref, minv_ref, psum_ref, best_ref, bidx_ref):
    k = pl.program_id(1)
    x = x_ref[...]
    w = w_ref[...]
    mm = lax.dot_general(x, w, (((1,), (1,)), ((), ())),
                         preferred_element_type=jnp.float32)
    x2 = jnp.sum(x * x, axis=1, keepdims=True)
    w2 = jnp.sum(w * w, axis=1, keepdims=True)
    d = (x2 + w2.reshape(1, _KB)) - 2.0 * mm
    m = jnp.min(d, axis=1, keepdims=True)
    iota = lax.broadcasted_iota(jnp.int32, (_TB, _KB), 1)
    li = jnp.min(jnp.where(d == m, iota, _K), axis=1, keepdims=True)
    gi = li + k * _KB

    @pl.when(k == 0)
    def _():
        best_ref[...] = m
        bidx_ref[...] = gi

    @pl.when(k > 0)
    def _():
        upd = m < best_ref[...]
        best_ref[...] = jnp.where(upd, m, best_ref[...])
        bidx_ref[...] = jnp.where(upd, gi, bidx_ref[...])

    @pl.when((pl.program_id(0) == 0) & (k == 0))
    def _():
        psum_ref[0, 0] = 0.0

    @pl.when(k == _NK - 1)
    def _():
        idx_ref[...] = bidx_ref[...]
        minv_ref[...] = best_ref[...]
        psum_ref[0, 0] += jnp.sum(best_ref[...])


_dist_call = pl.pallas_call(
    _dist_body,
    grid=(_NT, _NK),
    in_specs=[
        pl.BlockSpec((_TB, _D), lambda i, k: (i, 0)),
        pl.BlockSpec((_KB, _D), lambda i, k: (k, 0)),
    ],
    out_specs=[
        pl.BlockSpec((_TB, 1), lambda i, k: (i, 0)),
        pl.BlockSpec((_TB, 1), lambda i, k: (i, 0)),
        pl.BlockSpec((1, 1), lambda i, k: (0, 0), memory_space=pltpu.SMEM),
    ],
    out_shape=[
        jax.ShapeDtypeStruct((_N, 1), jnp.int32),
        jax.ShapeDtypeStruct((_N, 1), jnp.float32),
        jax.ShapeDtypeStruct((1, 1), jnp.float32),
    ],
    scratch_shapes=[
        pltpu.VMEM((_TB, 1), jnp.float32),
        pltpu.VMEM((_TB, 1), jnp.int32),
    ],
    compiler_params=pltpu.CompilerParams(
        dimension_semantics=("arbitrary", "arbitrary"),
    ),
)


@functools.cache
def _make_gather():
    @functools.partial(
        pl.kernel,
        mesh=plsc.VectorSubcoreMesh(core_axis_name="c", subcore_axis_name="s"),
        out_type=jax.ShapeDtypeStruct((_N, _D), jnp.float32),
        scratch_types=[
            pltpu.VMEM((_NCH, _CH), jnp.int32),
            pltpu.VMEM((_NCH, _CH, _D), jnp.float32),
            pltpu.SemaphoreType.DMA,
        ],
    )
    def _gather(w_hbm, idx_hbm, out_hbm, idx_v, rows_v, sem):
        wid = lax.axis_index("s") * 2 + lax.axis_index("c")
        base = wid * _BPW
        copies = []
        for j in range(_NCH):
            pltpu.sync_copy(idx_hbm.at[pl.ds(base + j * _CH, _CH)], idx_v.at[j])
            copies.append(
                pltpu.async_copy(w_hbm.at[idx_v.at[j]], rows_v.at[j], sem))
        for j in range(_NCH):
            copies[j].wait()
            pltpu.sync_copy(rows_v.at[j], out_hbm.at[pl.ds(base + j * _CH, _CH)])

    return _gather


def kernel(inputs, topic_embedding, theta, pretrain_vq, W):
    del theta
    Wsel = jnp.where(pretrain_vq != 0, topic_embedding, W)
    flat = inputs.reshape(_N, _D)
    idx2, minv2, psums = _dist_call(flat, Wsel)
    idx = idx2.reshape(_N)
    quantized = _make_gather()(Wsel, idx)
    loss_commit = psums[0, 0]
    loss = loss_commit * _CC + loss_commit
    quantized_st = quantized.reshape(inputs.shape)
    encoding_indices = idx.reshape(inputs.shape[:-1])
    min_distances = minv2.reshape(inputs.shape[:-1])
    return (quantized_st, loss, encoding_indices, min_distances, loss_commit)
